# baseline (speedup 1.0000x reference)
import functools

import jax
import jax.numpy as jnp
from jax import lax
from jax.experimental import pallas as pl
from jax.experimental.pallas import tpu as pltpu
from jax.experimental.pallas import tpu_sc as plsc

B_, H_, W_ = 4, 128, 128
NC, NS = 2, 16
NW = NC * NS
NCELL = B_ * H_ * W_
ROUNDS = 8
BUCK = NCELL // ROUNDS
LBUCK = 13
OWN = BUCK // NW
LOWN = 8
DUMP = OWN
PSHIFT = 17
PMASK = (1 << PSHIFT) - 1
LISTCAP = 4096
RING = 256


def _ceil_to(x, m):
    return (x + m - 1) // m * m


def _k1_body(PT, MT, MP, C,
             cb, cy, cx, mb, my, mx, strd,
             out_c, hpk, hmk, hcnt,
             cc0, cc1, cc2, pcell_v, mcell_v, strd_v, cnt_v,
             cppk, cmmk):
    PC = PT // 2
    MC = MT // 2
    c = lax.axis_index("c")
    s_id = lax.axis_index("s")
    w = c * NS + s_id
    pbase = w * PT
    mbase = w * MT
    iota = lax.broadcasted_iota(jnp.int32, (16,), 0)
    i32 = jnp.int32

    pltpu.sync_copy(strd.at[pl.ds(0, 16)], strd_v)
    stride_vec = strd_v[pl.ds(0, 16)]

    for q in range(2):
        off = q * PC
        pltpu.sync_copy(cb.at[pl.ds(pbase + off, PC)], cc0.at[pl.ds(0, PC)])
        pltpu.sync_copy(cy.at[pl.ds(pbase + off, PC)], cc1.at[pl.ds(0, PC)])
        pltpu.sync_copy(cx.at[pl.ds(pbase + off, PC)], cc2.at[pl.ds(0, PC)])

        def pcells(i, _, off=off):
            b = cc0[pl.ds(i * 16, 16)]
            y = cc1[pl.ds(i * 16, 16)]
            x = cc2[pl.ds(i * 16, 16)]
            yy = lax.div(y, stride_vec)
            xx = lax.div(x, stride_vec)
            ok = ((b >= 0) & (b < B_) & (yy >= 0) & (yy < H_)
                  & (xx >= 0) & (xx < W_))
            cell = (b * H_ + yy) * W_ + xx
            pcell_v[pl.ds(off + i * 16, 16)] = jnp.where(ok, cell, -1)
            return 0
        lax.fori_loop(0, PC // 16, pcells, 0)

    for q in range(2):
        off = q * MC
        pltpu.sync_copy(mb.at[pl.ds(mbase + off, MC)], cc0.at[pl.ds(0, MC)])
        pltpu.sync_copy(my.at[pl.ds(mbase + off, MC)], cc1.at[pl.ds(0, MC)])
        pltpu.sync_copy(mx.at[pl.ds(mbase + off, MC)], cc2.at[pl.ds(0, MC)])

        def mcells(i, _, off=off):
            cell = ((cc0[pl.ds(i * 16, 16)] * H_ + cc1[pl.ds(i * 16, 16)])
                    * W_ + cc2[pl.ds(i * 16, 16)])
            mcell_v[pl.ds(off + i * 16, 16)] = cell
            return 0
        lax.fori_loop(0, MC // 16, mcells, 0)

        pltpu.sync_copy(cc0.at[pl.ds(0, MC)],
                        out_c.at[pl.ds(mbase + off, MC)])

        def scale12(i, _):
            cc1[pl.ds(i * 16, 16)] = cc1[pl.ds(i * 16, 16)] * stride_vec
            cc2[pl.ds(i * 16, 16)] = cc2[pl.ds(i * 16, 16)] * stride_vec
            return 0
        lax.fori_loop(0, MC // 16, scale12, 0)
        pltpu.sync_copy(cc1.at[pl.ds(0, MC)],
                        out_c.at[pl.ds(MP + mbase + off, MC)])
        pltpu.sync_copy(cc2.at[pl.ds(0, MC)],
                        out_c.at[pl.ds(2 * MP + mbase + off, MC)])

    def partition(src_v, nvec, base, list_v, cntlane0):
        posv = jnp.zeros((16,), i32)
        for sl in range(ROUNDS):
            def scan(i, pv):
                cell = src_v[pl.ds(i * 16, 16)]
                hit = (cell >> LBUCK) == sl
                pos = pv + plsc.cumsum(hit.astype(i32)) - 1
                val = ((cell & (BUCK - 1)) << PSHIFT) | (base + i * 16 + iota)
                plsc.store_scatter(list_v, [pos], val, mask=hit)
                return pv + plsc.all_reduce_population_count(hit)
            posv2 = lax.fori_loop(0, nvec, scan, posv)
            plsc.store_scatter(cnt_v, [jnp.full((16,), cntlane0 + sl, i32)],
                              posv2 - posv)
            padv = (256 - (posv2 & 255)) & 255

            def pad16(k, _):
                idx = posv2 + k * 16 + iota
                plsc.store_scatter(list_v, [idx],
                                   jnp.full((16,), -1, i32),
                                   mask=(k * 16 + iota) < padv)
                return 0
            lax.fori_loop(0, 16, pad16, 0)
            posv = posv2 + padv
        return posv

    partition(pcell_v, PT // 16, pbase, cppk, 0)
    partition(mcell_v, MT // 16, mbase, cmmk, 16)

    pltpu.sync_copy(cppk, hpk.at[pl.ds(w * LISTCAP, LISTCAP)])
    pltpu.sync_copy(cmmk, hmk.at[pl.ds(w * LISTCAP, LISTCAP)])
    pltpu.sync_copy(cnt_v, hcnt.at[pl.ds(w * 32, 32)])


def _k2_body(MP, C,
             feat, mval, hpk, hmk, hcnt,
             out_f,
             cnts_in, inb, ring_g, ring_c, mring_g, mring_o, mring_c,
             frows, mvrows, dense, stamp):
    c = lax.axis_index("c")
    s_id = lax.axis_index("s")
    w = c * NS + s_id
    iota = lax.broadcasted_iota(jnp.int32, (16,), 0)
    ones16 = jnp.ones((16,), jnp.int32)
    i32 = jnp.int32
    NEG = jnp.int32(-(2**31) + 1)

    pltpu.sync_copy(hcnt, cnts_in)

    def fire_pt(_, fired):
        row = (fired >> 4) & 15
        pltpu.sync_copy(feat.at[ring_g.at[row]], frows)

        def acc_row(rr, _):
            base = (fired + rr) & (RING - 1)
            oc = ring_c[pl.ds(base, 16)][0]
            st = stamp[pl.ds(oc, 16)][0]
            for j in range(C // 16):
                d = dense[oc, pl.ds(j * 16, 16)]
                dense[oc, pl.ds(j * 16, 16)] = (
                    jnp.where(st != 0, d, jnp.float32(0.0))
                    + frows[rr, pl.ds(j * 16, 16)])
            plsc.store_scatter(stamp, [jnp.full((16,), oc, i32)], ones16)
            return 0
        lax.fori_loop(0, 16, acc_row, 0)
        return fired + 16

    def fire_mk(_, fired):
        row = (fired >> 4) & 15
        pltpu.sync_copy(mval.at[mring_g.at[row]], mvrows)

        def mul_row(rr, _):
            base = (fired + rr) & (RING - 1)
            oc = mring_c[pl.ds(base, 16)][0]
            st = stamp[pl.ds(oc, 16)][0]
            for j in range(C // 16):
                mvrows[rr, pl.ds(j * 16, 16)] = jnp.where(
                    st != 0,
                    dense[oc, pl.ds(j * 16, 16)]
                    * mvrows[rr, pl.ds(j * 16, 16)],
                    jnp.float32(0.0))
            return 0
        lax.fori_loop(0, 16, mul_row, 0)
        pltpu.sync_copy(mvrows, out_f.at[mring_o.at[row]])
        return fired + 16

    def round_body(r, _):
        def zst(k, _):
            stamp[pl.ds(k * 16, 16)] = jnp.zeros((16,), i32)
            return 0
        lax.fori_loop(0, (OWN + 32) // 16, zst, 0)

        def side(listref, lane0, ring_gr, ring_or, ring_cr, fire):
            def src_loop(t, carry):
                cv, fired = carry
                row0 = cnts_in[pl.ds(t * 32 + lane0, 16)]
                nchv = (row0 + 255) >> 8
                lo_ch = jnp.sum(jnp.where(iota < r, nchv, 0))
                cnt = jnp.max(jnp.where(iota == r, row0, NEG))
                nch = (cnt + 255) >> 8

                def chunk(q, c2):
                    cv, fired = c2
                    pltpu.sync_copy(
                        listref.at[pl.ds(
                            (t * (LISTCAP // 256) + lo_ch + q) * 256, 256)],
                        inb)

                    def scan16(i, c3):
                        cv, fired = c3
                        pk = inb[pl.ds(i * 16, 16)]
                        lcell = pk >> PSHIFT
                        gidx = pk & PMASK
                        hit = (lcell >> LOWN) == w
                        pos = cv + plsc.cumsum(hit.astype(i32)) - 1
                        plsc.store_scatter(ring_gr,
                                           [(pos >> 4) & 15, pos & 15],
                                           gidx, mask=hit)
                        if ring_or is not None:
                            plsc.store_scatter(ring_or,
                                               [(pos >> 4) & 15, pos & 15],
                                               gidx, mask=hit)
                        plsc.store_scatter(ring_cr, [pos & (RING - 1)],
                                           lcell & (OWN - 1), mask=hit)
                        cv = cv + plsc.all_reduce_population_count(hit)
                        nf = (jnp.max(cv) - fired) >> 4
                        fired = lax.fori_loop(0, nf, fire, fired)
                        return (cv, fired)
                    return lax.fori_loop(0, 16, scan16, (cv, fired))
                return lax.fori_loop(0, nch, chunk, (cv, fired))
            cv, fired = lax.fori_loop(0, NW, src_loop,
                                      (jnp.zeros((16,), i32), i32(0)))
            mcnt = jnp.max(cv)
            padmask = (fired + iota) >= mcnt
            rowsp = jnp.full((16,), (fired >> 4) & 15, i32)
            plsc.store_scatter(ring_gr, [rowsp, iota],
                               jnp.zeros((16,), i32), mask=padmask)
            if ring_or is not None:
                plsc.store_scatter(ring_or, [rowsp, iota],
                                   jnp.full((16,), MP, i32), mask=padmask)
            plsc.store_scatter(ring_cr, [(fired & (RING - 1)) + iota],
                               jnp.full((16,), DUMP, i32), mask=padmask)
            lax.fori_loop(0, (mcnt - fired + 15) >> 4, fire, fired)

        side(hpk, 0, ring_g, None, ring_c, fire_pt)
        side(hmk, 16, mring_g, mring_o, mring_c, fire_mk)
        return 0

    lax.fori_loop(0, ROUNDS, round_body, 0)


def kernel(features, mask_values, coords_b, coords_y, coords_x,
           mask_b, mask_y, mask_x, stride):
    N, C = features.shape
    M = mask_values.shape[0]
    PT = _ceil_to(-(-N // NW), 32)
    MT = _ceil_to(-(-M // NW), 32)
    NP = NW * PT
    MP = NW * MT
    BIG = jnp.int32(1 << 20)

    cb = jnp.pad(coords_b, (0, NP - N))
    cy = jnp.pad(coords_y, (0, NP - N), constant_values=BIG)
    cx = jnp.pad(coords_x, (0, NP - N))
    mb = jnp.pad(mask_b, (0, MP - M))
    my = jnp.pad(mask_y, (0, MP - M), constant_values=BIG)
    mx = jnp.pad(mask_x, (0, MP - M))
    strd = jnp.full((16,), stride, jnp.int32)

    mesh = plsc.VectorSubcoreMesh(core_axis_name="c", subcore_axis_name="s")
    cp = pltpu.CompilerParams(needs_layout_passes=False)

    out_c, hpk, hmk, hcnt = pl.kernel(
        functools.partial(_k1_body, PT, MT, MP, C),
        out_type=(
            jax.ShapeDtypeStruct((3 * MP,), jnp.int32),
            jax.ShapeDtypeStruct((NW * LISTCAP,), jnp.int32),
            jax.ShapeDtypeStruct((NW * LISTCAP,), jnp.int32),
            jax.ShapeDtypeStruct((NW * 32,), jnp.int32),
        ),
        mesh=mesh,
        compiler_params=cp,
        scratch_types=[
            pltpu.VMEM((PT // 2,), jnp.int32),
            pltpu.VMEM((PT // 2,), jnp.int32),
            pltpu.VMEM((PT // 2,), jnp.int32),
            pltpu.VMEM((PT,), jnp.int32),
            pltpu.VMEM((MT,), jnp.int32),
            pltpu.VMEM((16,), jnp.int32),
            pltpu.VMEM((32,), jnp.int32),
            pltpu.VMEM((LISTCAP,), jnp.int32),
            pltpu.VMEM((LISTCAP,), jnp.int32),
        ],
    )(cb, cy, cx, mb, my, mx, strd)

    out_f, = pl.kernel(
        functools.partial(_k2_body, MP, C),
        out_type=(
            jax.ShapeDtypeStruct((MP + 8, C), jnp.float32),
        ),
        mesh=mesh,
        compiler_params=cp,
        scratch_types=[
            pltpu.VMEM((NW * 32,), jnp.int32),
            pltpu.VMEM((256,), jnp.int32),
            pltpu.VMEM((16, 16), jnp.int32),
            pltpu.VMEM((RING + 16,), jnp.int32),
            pltpu.VMEM((16, 16), jnp.int32),
            pltpu.VMEM((16, 16), jnp.int32),
            pltpu.VMEM((RING + 16,), jnp.int32),
            pltpu.VMEM((16, C), jnp.float32),
            pltpu.VMEM((16, C), jnp.float32),
            pltpu.VMEM((OWN + 8, C), jnp.float32),
            pltpu.VMEM((OWN + 32,), jnp.int32),
        ],
    )(features, mask_values, hpk, hmk, hcnt)

    out_feats = out_f[:M]
    out_coords = out_c.reshape(3, MP)[:, :M].T
    return out_feats, out_coords


# speedup vs baseline: 1.3363x; 1.3363x over previous
import functools

import jax
import jax.numpy as jnp
from jax import lax
from jax.experimental import pallas as pl
from jax.experimental.pallas import tpu as pltpu
from jax.experimental.pallas import tpu_sc as plsc

B_, H_, W_ = 4, 128, 128
NC, NS = 2, 16
NW = NC * NS
NCELL = B_ * H_ * W_
ROUNDS = 8
BUCK = NCELL // ROUNDS
LBUCK = 13
OWN = BUCK // NW
LOWN = 8
DUMP = OWN
PSHIFT = 17
PMASK = (1 << PSHIFT) - 1
LISTCAP = 4096
RING = 256


def _ceil_to(x, m):
    return (x + m - 1) // m * m


def _k1_body(PT, MT, MP, C,
             cb, cy, cx, mb, my, mx, strd,
             out_c, hpk, hmk, hcnt,
             cc0, cc1, cc2, pcell_v, mcell_v, strd_v, cnt_v,
             cppk, cmmk):
    PC = PT // 2
    MC = MT // 2
    c = lax.axis_index("c")
    s_id = lax.axis_index("s")
    w = c * NS + s_id
    pbase = w * PT
    mbase = w * MT
    iota = lax.broadcasted_iota(jnp.int32, (16,), 0)
    i32 = jnp.int32

    pltpu.sync_copy(strd.at[pl.ds(0, 16)], strd_v)
    stride_vec = strd_v[pl.ds(0, 16)]

    for q in range(2):
        off = q * PC
        pltpu.sync_copy(cb.at[pl.ds(pbase + off, PC)], cc0.at[pl.ds(0, PC)])
        pltpu.sync_copy(cy.at[pl.ds(pbase + off, PC)], cc1.at[pl.ds(0, PC)])
        pltpu.sync_copy(cx.at[pl.ds(pbase + off, PC)], cc2.at[pl.ds(0, PC)])

        def pcells(i, _, off=off):
            b = cc0[pl.ds(i * 16, 16)]
            y = cc1[pl.ds(i * 16, 16)]
            x = cc2[pl.ds(i * 16, 16)]
            yy = lax.div(y, stride_vec)
            xx = lax.div(x, stride_vec)
            ok = ((b >= 0) & (b < B_) & (yy >= 0) & (yy < H_)
                  & (xx >= 0) & (xx < W_))
            cell = (b * H_ + yy) * W_ + xx
            pcell_v[pl.ds(off + i * 16, 16)] = jnp.where(ok, cell, -1)
            return 0
        lax.fori_loop(0, PC // 16, pcells, 0)

    for q in range(2):
        off = q * MC
        pltpu.sync_copy(mb.at[pl.ds(mbase + off, MC)], cc0.at[pl.ds(0, MC)])
        pltpu.sync_copy(my.at[pl.ds(mbase + off, MC)], cc1.at[pl.ds(0, MC)])
        pltpu.sync_copy(mx.at[pl.ds(mbase + off, MC)], cc2.at[pl.ds(0, MC)])

        def mcells(i, _, off=off):
            cell = ((cc0[pl.ds(i * 16, 16)] * H_ + cc1[pl.ds(i * 16, 16)])
                    * W_ + cc2[pl.ds(i * 16, 16)])
            mcell_v[pl.ds(off + i * 16, 16)] = cell
            return 0
        lax.fori_loop(0, MC // 16, mcells, 0)

        pltpu.sync_copy(cc0.at[pl.ds(0, MC)],
                        out_c.at[pl.ds(mbase + off, MC)])

        def scale12(i, _):
            cc1[pl.ds(i * 16, 16)] = cc1[pl.ds(i * 16, 16)] * stride_vec
            cc2[pl.ds(i * 16, 16)] = cc2[pl.ds(i * 16, 16)] * stride_vec
            return 0
        lax.fori_loop(0, MC // 16, scale12, 0)
        pltpu.sync_copy(cc1.at[pl.ds(0, MC)],
                        out_c.at[pl.ds(MP + mbase + off, MC)])
        pltpu.sync_copy(cc2.at[pl.ds(0, MC)],
                        out_c.at[pl.ds(2 * MP + mbase + off, MC)])

    def partition(src_v, nvec, base, list_v, cntlane0):
        posv = jnp.zeros((16,), i32)
        for sl in range(ROUNDS):
            def scan(i, pv):
                cell = src_v[pl.ds(i * 16, 16)]
                hit = (cell >> LBUCK) == sl
                pos = pv + plsc.cumsum(hit.astype(i32)) - 1
                val = ((cell & (BUCK - 1)) << PSHIFT) | (base + i * 16 + iota)
                plsc.store_scatter(list_v, [pos], val, mask=hit)
                return pv + plsc.all_reduce_population_count(hit)
            posv2 = lax.fori_loop(0, nvec, scan, posv)
            plsc.store_scatter(cnt_v, [jnp.full((16,), cntlane0 + sl, i32)],
                              posv2 - posv)
            padv = (256 - (posv2 & 255)) & 255

            def pad16(k, _):
                idx = posv2 + k * 16 + iota
                plsc.store_scatter(list_v, [idx],
                                   jnp.full((16,), -1, i32),
                                   mask=(k * 16 + iota) < padv)
                return 0
            lax.fori_loop(0, 16, pad16, 0)
            posv = posv2 + padv
        return posv

    partition(pcell_v, PT // 16, pbase, cppk, 0)
    partition(mcell_v, MT // 16, mbase, cmmk, 16)

    pltpu.sync_copy(cppk, hpk.at[pl.ds(w * LISTCAP, LISTCAP)])
    pltpu.sync_copy(cmmk, hmk.at[pl.ds(w * LISTCAP, LISTCAP)])
    pltpu.sync_copy(cnt_v, hcnt.at[pl.ds(w * 32, 32)])


def _k2_body(MP, C,
             feat, mval, hpk, hmk, hcnt,
             out_f,
             cnts_in, inb, inb2, ring_g, ring_c, mring_g, mring_o, mring_c,
             frows, mvrows, dense, stamp, sem):
    c = lax.axis_index("c")
    s_id = lax.axis_index("s")
    w = c * NS + s_id
    iota = lax.broadcasted_iota(jnp.int32, (16,), 0)
    ones16 = jnp.ones((16,), jnp.int32)
    i32 = jnp.int32
    NEG = jnp.int32(-(2**31) + 1)
    FB = 32

    pltpu.sync_copy(hcnt, cnts_in)

    def fire_pt(_, fired):
        row = (fired >> 5) & 7
        pltpu.sync_copy(feat.at[ring_g.at[row]], frows)

        def acc_row(rr, _):
            base = (fired + rr) & (RING - 1)
            oc = ring_c[pl.ds(base, 16)][0]
            st = stamp[pl.ds(oc, 16)][0]
            for j in range(C // 16):
                d = dense[oc, pl.ds(j * 16, 16)]
                dense[oc, pl.ds(j * 16, 16)] = (
                    jnp.where(st != 0, d, jnp.float32(0.0))
                    + frows[rr, pl.ds(j * 16, 16)])
            plsc.store_scatter(stamp, [jnp.full((16,), oc, i32)], ones16)
            return 0
        lax.fori_loop(0, FB, acc_row, 0)
        return fired + FB

    def fire_mk(_, fired):
        row = (fired >> 5) & 7
        pltpu.sync_copy(mval.at[mring_g.at[row]], mvrows)

        def mul_row(rr, _):
            base = (fired + rr) & (RING - 1)
            oc = mring_c[pl.ds(base, 16)][0]
            st = stamp[pl.ds(oc, 16)][0]
            for j in range(C // 16):
                mvrows[rr, pl.ds(j * 16, 16)] = jnp.where(
                    st != 0,
                    dense[oc, pl.ds(j * 16, 16)]
                    * mvrows[rr, pl.ds(j * 16, 16)],
                    jnp.float32(0.0))
            return 0
        lax.fori_loop(0, FB, mul_row, 0)
        pltpu.sync_copy(mvrows, out_f.at[mring_o.at[row]])
        return fired + FB

    def round_body(r, _):
        def zst(k, _):
            stamp[pl.ds(k * 16, 16)] = jnp.zeros((16,), i32)
            return 0
        lax.fori_loop(0, (OWN + 32) // 16, zst, 0)

        def side(listref, lane0, ring_gr, ring_or, ring_cr, fire):
            def seg(t):
                row0 = cnts_in[pl.ds(t * 32 + lane0, 16)]
                nchv = (row0 + 255) >> 8
                lo_ch = jnp.sum(jnp.where(iota < r, nchv, 0))
                cnt = jnp.max(jnp.where(iota == r, row0, NEG))
                return t * (LISTCAP // 256) + lo_ch, cnt

            def scan_chunk(buf, cnt_left, cv, fired):
                nv = (jnp.minimum(cnt_left, 256) + 15) >> 4

                def scan16(i, c3):
                    cv, fired = c3
                    pk = buf[pl.ds(i * 16, 16)]
                    lcell = pk >> PSHIFT
                    gidx = pk & PMASK
                    hit = (lcell >> LOWN) == w
                    pos = cv + plsc.cumsum(hit.astype(i32)) - 1
                    plsc.store_scatter(ring_gr,
                                       [(pos >> 5) & 7, pos & 31],
                                       gidx, mask=hit)
                    if ring_or is not None:
                        plsc.store_scatter(ring_or,
                                           [(pos >> 5) & 7, pos & 31],
                                           gidx, mask=hit)
                    plsc.store_scatter(ring_cr, [pos & (RING - 1)],
                                       lcell & (OWN - 1), mask=hit)
                    cv = cv + plsc.all_reduce_population_count(hit)
                    nf = (jnp.max(cv) - fired) >> 5
                    fired = lax.fori_loop(0, nf, fire, fired)
                    return (cv, fired)
                return lax.fori_loop(0, nv, scan16, (cv, fired))

            ch0, _ = seg(0)
            pltpu.async_copy(listref.at[pl.ds(ch0 * 256, 256)],
                             inb.at[pl.ds(0, 256)], sem)

            def src_loop(t, carry):
                cv, fired = carry
                _, cnt = seg(t)
                nch = (cnt + 255) >> 8
                pltpu.make_async_copy(listref.at[pl.ds(0, 256)],
                                      inb.at[pl.ds((t & 1) * 256, 256)],
                                      sem).wait()
                tn = jnp.minimum(t + 1, NW - 1)
                chn, _ = seg(tn)
                pltpu.async_copy(listref.at[pl.ds(chn * 256, 256)],
                                 inb.at[pl.ds(((t + 1) & 1) * 256, 256)],
                                 sem)
                cv, fired = scan_chunk(inb.at[pl.ds((t & 1) * 256, 256)],
                                       cnt, cv, fired)

                def chunk(q, c2):
                    cv, fired = c2
                    ch, _ = seg(t)
                    pltpu.sync_copy(listref.at[pl.ds((ch + q) * 256, 256)],
                                    inb2)
                    return scan_chunk(inb2, cnt - q * 256, cv, fired)
                return lax.fori_loop(1, nch, chunk, (cv, fired))
            cv, fired = lax.fori_loop(0, NW, src_loop,
                                      (jnp.zeros((16,), i32), i32(0)))
            pltpu.make_async_copy(listref.at[pl.ds(0, 256)],
                                  inb.at[pl.ds(0, 256)], sem).wait()
            mcnt = jnp.max(cv)
            rowsp = jnp.full((16,), (fired >> 5) & 7, i32)
            for k in range(FB // 16):
                padmask = (fired + k * 16 + iota) >= mcnt
                colv = (fired + k * 16 + iota) & 31
                plsc.store_scatter(ring_gr, [rowsp, colv],
                                   jnp.zeros((16,), i32), mask=padmask)
                if ring_or is not None:
                    plsc.store_scatter(ring_or, [rowsp, colv],
                                       jnp.full((16,), MP, i32),
                                       mask=padmask)
                plsc.store_scatter(ring_cr,
                                   [(fired & (RING - 1)) + k * 16 + iota],
                                   jnp.full((16,), DUMP, i32), mask=padmask)
            lax.fori_loop(0, (mcnt - fired + FB - 1) >> 5, fire, fired)

        side(hpk, 0, ring_g, None, ring_c, fire_pt)
        side(hmk, 16, mring_g, mring_o, mring_c, fire_mk)
        return 0

    lax.fori_loop(0, ROUNDS, round_body, 0)


def kernel(features, mask_values, coords_b, coords_y, coords_x,
           mask_b, mask_y, mask_x, stride):
    N, C = features.shape
    M = mask_values.shape[0]
    PT = _ceil_to(-(-N // NW), 32)
    MT = _ceil_to(-(-M // NW), 32)
    NP = NW * PT
    MP = NW * MT
    BIG = jnp.int32(1 << 20)

    cb = jnp.pad(coords_b, (0, NP - N))
    cy = jnp.pad(coords_y, (0, NP - N), constant_values=BIG)
    cx = jnp.pad(coords_x, (0, NP - N))
    mb = jnp.pad(mask_b, (0, MP - M))
    my = jnp.pad(mask_y, (0, MP - M), constant_values=BIG)
    mx = jnp.pad(mask_x, (0, MP - M))
    strd = jnp.full((16,), stride, jnp.int32)

    mesh = plsc.VectorSubcoreMesh(core_axis_name="c", subcore_axis_name="s")
    cp = pltpu.CompilerParams(needs_layout_passes=False)

    out_c, hpk, hmk, hcnt = pl.kernel(
        functools.partial(_k1_body, PT, MT, MP, C),
        out_type=(
            jax.ShapeDtypeStruct((3 * MP,), jnp.int32),
            jax.ShapeDtypeStruct((NW * LISTCAP,), jnp.int32),
            jax.ShapeDtypeStruct((NW * LISTCAP,), jnp.int32),
            jax.ShapeDtypeStruct((NW * 32,), jnp.int32),
        ),
        mesh=mesh,
        compiler_params=cp,
        scratch_types=[
            pltpu.VMEM((PT // 2,), jnp.int32),
            pltpu.VMEM((PT // 2,), jnp.int32),
            pltpu.VMEM((PT // 2,), jnp.int32),
            pltpu.VMEM((PT,), jnp.int32),
            pltpu.VMEM((MT,), jnp.int32),
            pltpu.VMEM((16,), jnp.int32),
            pltpu.VMEM((32,), jnp.int32),
            pltpu.VMEM((LISTCAP,), jnp.int32),
            pltpu.VMEM((LISTCAP,), jnp.int32),
        ],
    )(cb, cy, cx, mb, my, mx, strd)

    out_f, = pl.kernel(
        functools.partial(_k2_body, MP, C),
        out_type=(
            jax.ShapeDtypeStruct((MP + 8, C), jnp.float32),
        ),
        mesh=mesh,
        compiler_params=cp,
        scratch_types=[
            pltpu.VMEM((NW * 32,), jnp.int32),
            pltpu.VMEM((512,), jnp.int32),
            pltpu.VMEM((256,), jnp.int32),
            pltpu.VMEM((8, 32), jnp.int32),
            pltpu.VMEM((RING + 16,), jnp.int32),
            pltpu.VMEM((8, 32), jnp.int32),
            pltpu.VMEM((8, 32), jnp.int32),
            pltpu.VMEM((RING + 16,), jnp.int32),
            pltpu.VMEM((32, C), jnp.float32),
            pltpu.VMEM((32, C), jnp.float32),
            pltpu.VMEM((OWN + 8, C), jnp.float32),
            pltpu.VMEM((OWN + 32,), jnp.int32),
            pltpu.SemaphoreType.DMA,
        ],
    )(features, mask_values, hpk, hmk, hcnt)

    out_feats = out_f[:M]
    out_coords = out_c.reshape(3, MP)[:, :M].T
    return out_feats, out_coords


# speedup vs baseline: 1.5569x; 1.1651x over previous
import functools

import jax
import jax.numpy as jnp
from jax import lax
from jax.experimental import pallas as pl
from jax.experimental.pallas import tpu as pltpu
from jax.experimental.pallas import tpu_sc as plsc

B_, H_, W_ = 4, 128, 128
NC, NS = 2, 16
NW = NC * NS
NCELL = B_ * H_ * W_
ROUNDS = 8
BUCK = NCELL // ROUNDS
LBUCK = 13
OWN = BUCK // NW
LOWN = 8
DUMP = OWN
PSHIFT = 17
PMASK = (1 << PSHIFT) - 1
LISTCAP = 4096
RING = 512


def _ceil_to(x, m):
    return (x + m - 1) // m * m


def _k1_body(PT, MT, MP, C,
             cb, cy, cx, mb, my, mx, strd,
             out_c, hpk, hmk, hcnt,
             cc0, cc1, cc2, pcell_v, mcell_v, strd_v, cnt_v,
             cppk, cmmk):
    PC = PT // 2
    MC = MT // 2
    c = lax.axis_index("c")
    s_id = lax.axis_index("s")
    w = c * NS + s_id
    pbase = w * PT
    mbase = w * MT
    iota = lax.broadcasted_iota(jnp.int32, (16,), 0)
    i32 = jnp.int32

    pltpu.sync_copy(strd.at[pl.ds(0, 16)], strd_v)
    stride_vec = strd_v[pl.ds(0, 16)]

    for q in range(2):
        off = q * PC
        pltpu.sync_copy(cb.at[pl.ds(pbase + off, PC)], cc0.at[pl.ds(0, PC)])
        pltpu.sync_copy(cy.at[pl.ds(pbase + off, PC)], cc1.at[pl.ds(0, PC)])
        pltpu.sync_copy(cx.at[pl.ds(pbase + off, PC)], cc2.at[pl.ds(0, PC)])

        def pcells(i, _, off=off):
            b = cc0[pl.ds(i * 16, 16)]
            y = cc1[pl.ds(i * 16, 16)]
            x = cc2[pl.ds(i * 16, 16)]
            yy = lax.div(y, stride_vec)
            xx = lax.div(x, stride_vec)
            ok = ((b >= 0) & (b < B_) & (yy >= 0) & (yy < H_)
                  & (xx >= 0) & (xx < W_))
            cell = (b * H_ + yy) * W_ + xx
            pcell_v[pl.ds(off + i * 16, 16)] = jnp.where(ok, cell, -1)
            return 0
        lax.fori_loop(0, PC // 16, pcells, 0)

    for q in range(2):
        off = q * MC
        pltpu.sync_copy(mb.at[pl.ds(mbase + off, MC)], cc0.at[pl.ds(0, MC)])
        pltpu.sync_copy(my.at[pl.ds(mbase + off, MC)], cc1.at[pl.ds(0, MC)])
        pltpu.sync_copy(mx.at[pl.ds(mbase + off, MC)], cc2.at[pl.ds(0, MC)])

        def mcells(i, _, off=off):
            cell = ((cc0[pl.ds(i * 16, 16)] * H_ + cc1[pl.ds(i * 16, 16)])
                    * W_ + cc2[pl.ds(i * 16, 16)])
            mcell_v[pl.ds(off + i * 16, 16)] = cell
            return 0
        lax.fori_loop(0, MC // 16, mcells, 0)

        pltpu.sync_copy(cc0.at[pl.ds(0, MC)],
                        out_c.at[pl.ds(mbase + off, MC)])

        def scale12(i, _):
            cc1[pl.ds(i * 16, 16)] = cc1[pl.ds(i * 16, 16)] * stride_vec
            cc2[pl.ds(i * 16, 16)] = cc2[pl.ds(i * 16, 16)] * stride_vec
            return 0
        lax.fori_loop(0, MC // 16, scale12, 0)
        pltpu.sync_copy(cc1.at[pl.ds(0, MC)],
                        out_c.at[pl.ds(MP + mbase + off, MC)])
        pltpu.sync_copy(cc2.at[pl.ds(0, MC)],
                        out_c.at[pl.ds(2 * MP + mbase + off, MC)])

    def partition(src_v, nvec, base, list_v, cntlane0):
        posv = jnp.zeros((16,), i32)
        for sl in range(ROUNDS):
            def scan(i, pv):
                cell = src_v[pl.ds(i * 16, 16)]
                hit = (cell >> LBUCK) == sl
                pos = pv + plsc.cumsum(hit.astype(i32)) - 1
                val = ((cell & (BUCK - 1)) << PSHIFT) | (base + i * 16 + iota)
                plsc.store_scatter(list_v, [pos], val, mask=hit)
                return pv + plsc.all_reduce_population_count(hit)
            posv2 = lax.fori_loop(0, nvec, scan, posv)
            plsc.store_scatter(cnt_v, [jnp.full((16,), cntlane0 + sl, i32)],
                              posv2 - posv)
            padv = (256 - (posv2 & 255)) & 255

            def pad16(k, _):
                idx = posv2 + k * 16 + iota
                plsc.store_scatter(list_v, [idx],
                                   jnp.full((16,), -1, i32),
                                   mask=(k * 16 + iota) < padv)
                return 0
            lax.fori_loop(0, 16, pad16, 0)
            posv = posv2 + padv
        return posv

    partition(pcell_v, PT // 16, pbase, cppk, 0)
    partition(mcell_v, MT // 16, mbase, cmmk, 16)

    pltpu.sync_copy(cppk, hpk.at[pl.ds(w * LISTCAP, LISTCAP)])
    pltpu.sync_copy(cmmk, hmk.at[pl.ds(w * LISTCAP, LISTCAP)])
    pltpu.sync_copy(cnt_v, hcnt.at[pl.ds(w * 32, 32)])


def _k2_body(MP, C,
             feat, mval, hpk, hmk, hcnt,
             out_f,
             cnts_in, inb, inb2, ring_g, ring_c, mring_g, mring_o, mring_c,
             frows, mvrows, dense, stamp, sem):
    c = lax.axis_index("c")
    s_id = lax.axis_index("s")
    w = c * NS + s_id
    iota = lax.broadcasted_iota(jnp.int32, (16,), 0)
    ones16 = jnp.ones((16,), jnp.int32)
    i32 = jnp.int32
    NEG = jnp.int32(-(2**31) + 1)
    FB = 32

    pltpu.sync_copy(hcnt, cnts_in)

    def fire_pt(_, fired):
        row = (fired >> 5) & 15
        pltpu.sync_copy(feat.at[ring_g.at[row]], frows)

        def acc_row(rr, _):
            base = (fired + rr) & (RING - 1)
            oc = ring_c[pl.ds(base, 16)][0]
            st = stamp[pl.ds(oc, 16)][0]
            for j in range(C // 16):
                d = dense[oc, pl.ds(j * 16, 16)]
                dense[oc, pl.ds(j * 16, 16)] = (
                    jnp.where(st != 0, d, jnp.float32(0.0))
                    + frows[rr, pl.ds(j * 16, 16)])
            plsc.store_scatter(stamp, [jnp.full((16,), oc, i32)], ones16)
            return 0
        lax.fori_loop(0, FB, acc_row, 0)
        return fired + FB

    def fire_mk(_, fired):
        row = (fired >> 5) & 15
        pltpu.sync_copy(mval.at[mring_g.at[row]], mvrows)

        def mul_row(rr, _):
            base = (fired + rr) & (RING - 1)
            oc = mring_c[pl.ds(base, 16)][0]
            st = stamp[pl.ds(oc, 16)][0]
            for j in range(C // 16):
                mvrows[rr, pl.ds(j * 16, 16)] = jnp.where(
                    st != 0,
                    dense[oc, pl.ds(j * 16, 16)]
                    * mvrows[rr, pl.ds(j * 16, 16)],
                    jnp.float32(0.0))
            return 0
        lax.fori_loop(0, FB, mul_row, 0)
        pltpu.sync_copy(mvrows, out_f.at[mring_o.at[row]])
        return fired + FB

    def round_body(r, _):
        def zst(k, _):
            stamp[pl.ds(k * 16, 16)] = jnp.zeros((16,), i32)
            return 0
        lax.fori_loop(0, (OWN + 32) // 16, zst, 0)

        def side(listref, lane0, ring_gr, ring_or, ring_cr, fire):
            def seg(t):
                row0 = cnts_in[pl.ds(t * 32 + lane0, 16)]
                nchv = (row0 + 255) >> 8
                lo_ch = jnp.sum(jnp.where(iota < r, nchv, 0))
                cnt = jnp.max(jnp.where(iota == r, row0, NEG))
                return t * (LISTCAP // 256) + lo_ch, cnt

            def scan_chunk(buf, cnt_left, cv, fired):
                nv = (jnp.minimum(cnt_left, 256) + 15) >> 4

                def scan16(i, cv):
                    pk = buf[pl.ds(i * 16, 16)]
                    lcell = pk >> PSHIFT
                    gidx = pk & PMASK
                    hit = (lcell >> LOWN) == w
                    pos = cv + plsc.cumsum(hit.astype(i32)) - 1
                    plsc.store_scatter(ring_gr,
                                       [(pos >> 5) & 15, pos & 31],
                                       gidx, mask=hit)
                    if ring_or is not None:
                        plsc.store_scatter(ring_or,
                                           [(pos >> 5) & 15, pos & 31],
                                           gidx, mask=hit)
                    plsc.store_scatter(ring_cr, [pos & (RING - 1)],
                                       lcell & (OWN - 1), mask=hit)
                    return cv + plsc.all_reduce_population_count(hit)
                cv = lax.fori_loop(0, nv, scan16, cv)
                nf = (jnp.max(cv) - fired) >> 5
                fired = lax.fori_loop(0, nf, fire, fired)
                return (cv, fired)

            for tp in range(4):
                chp, _ = seg(tp)
                pltpu.async_copy(listref.at[pl.ds(chp * 256, 256)],
                                 inb.at[pl.ds(tp * 256, 256)], sem)

            def src_loop(t, carry):
                cv, fired = carry
                _, cnt = seg(t)
                nch = (cnt + 255) >> 8
                pltpu.make_async_copy(listref.at[pl.ds(0, 256)],
                                      inb.at[pl.ds((t & 3) * 256, 256)],
                                      sem).wait()
                cv, fired = scan_chunk(inb.at[pl.ds((t & 3) * 256, 256)],
                                       cnt, cv, fired)
                tn = jnp.minimum(t + 4, NW - 1)
                chn, _ = seg(tn)
                pltpu.async_copy(listref.at[pl.ds(chn * 256, 256)],
                                 inb.at[pl.ds((t & 3) * 256, 256)], sem)

                def chunk(q, c2):
                    cv, fired = c2
                    ch, _ = seg(t)
                    pltpu.sync_copy(listref.at[pl.ds((ch + q) * 256, 256)],
                                    inb2)
                    return scan_chunk(inb2, cnt - q * 256, cv, fired)
                return lax.fori_loop(1, nch, chunk, (cv, fired))
            cv, fired = lax.fori_loop(0, NW, src_loop,
                                      (jnp.zeros((16,), i32), i32(0)))
            for tp in range(4):
                pltpu.make_async_copy(listref.at[pl.ds(0, 256)],
                                      inb.at[pl.ds(tp * 256, 256)],
                                      sem).wait()
            mcnt = jnp.max(cv)
            rowsp = jnp.full((16,), (fired >> 5) & 15, i32)
            for k in range(FB // 16):
                padmask = (fired + k * 16 + iota) >= mcnt
                colv = (fired + k * 16 + iota) & 31
                plsc.store_scatter(ring_gr, [rowsp, colv],
                                   jnp.zeros((16,), i32), mask=padmask)
                if ring_or is not None:
                    plsc.store_scatter(ring_or, [rowsp, colv],
                                       jnp.full((16,), MP, i32),
                                       mask=padmask)
                plsc.store_scatter(ring_cr,
                                   [(fired & (RING - 1)) + k * 16 + iota],
                                   jnp.full((16,), DUMP, i32), mask=padmask)
            lax.fori_loop(0, (mcnt - fired + FB - 1) >> 5, fire, fired)

        side(hpk, 0, ring_g, None, ring_c, fire_pt)
        side(hmk, 16, mring_g, mring_o, mring_c, fire_mk)
        return 0

    lax.fori_loop(0, ROUNDS, round_body, 0)


def kernel(features, mask_values, coords_b, coords_y, coords_x,
           mask_b, mask_y, mask_x, stride):
    N, C = features.shape
    M = mask_values.shape[0]
    PT = _ceil_to(-(-N // NW), 32)
    MT = _ceil_to(-(-M // NW), 32)
    NP = NW * PT
    MP = NW * MT
    BIG = jnp.int32(1 << 20)

    cb = jnp.pad(coords_b, (0, NP - N))
    cy = jnp.pad(coords_y, (0, NP - N), constant_values=BIG)
    cx = jnp.pad(coords_x, (0, NP - N))
    mb = jnp.pad(mask_b, (0, MP - M))
    my = jnp.pad(mask_y, (0, MP - M), constant_values=BIG)
    mx = jnp.pad(mask_x, (0, MP - M))
    strd = jnp.full((16,), stride, jnp.int32)

    mesh = plsc.VectorSubcoreMesh(core_axis_name="c", subcore_axis_name="s")
    cp = pltpu.CompilerParams(needs_layout_passes=False)

    out_c, hpk, hmk, hcnt = pl.kernel(
        functools.partial(_k1_body, PT, MT, MP, C),
        out_type=(
            jax.ShapeDtypeStruct((3 * MP,), jnp.int32),
            jax.ShapeDtypeStruct((NW * LISTCAP,), jnp.int32),
            jax.ShapeDtypeStruct((NW * LISTCAP,), jnp.int32),
            jax.ShapeDtypeStruct((NW * 32,), jnp.int32),
        ),
        mesh=mesh,
        compiler_params=cp,
        scratch_types=[
            pltpu.VMEM((PT // 2,), jnp.int32),
            pltpu.VMEM((PT // 2,), jnp.int32),
            pltpu.VMEM((PT // 2,), jnp.int32),
            pltpu.VMEM((PT,), jnp.int32),
            pltpu.VMEM((MT,), jnp.int32),
            pltpu.VMEM((16,), jnp.int32),
            pltpu.VMEM((32,), jnp.int32),
            pltpu.VMEM((LISTCAP,), jnp.int32),
            pltpu.VMEM((LISTCAP,), jnp.int32),
        ],
    )(cb, cy, cx, mb, my, mx, strd)

    out_f, = pl.kernel(
        functools.partial(_k2_body, MP, C),
        out_type=(
            jax.ShapeDtypeStruct((MP + 8, C), jnp.float32),
        ),
        mesh=mesh,
        compiler_params=cp,
        scratch_types=[
            pltpu.VMEM((NW * 32,), jnp.int32),
            pltpu.VMEM((1024,), jnp.int32),
            pltpu.VMEM((256,), jnp.int32),
            pltpu.VMEM((16, 32), jnp.int32),
            pltpu.VMEM((RING + 16,), jnp.int32),
            pltpu.VMEM((16, 32), jnp.int32),
            pltpu.VMEM((16, 32), jnp.int32),
            pltpu.VMEM((RING + 16,), jnp.int32),
            pltpu.VMEM((32, C), jnp.float32),
            pltpu.VMEM((32, C), jnp.float32),
            pltpu.VMEM((OWN + 8, C), jnp.float32),
            pltpu.VMEM((OWN + 32,), jnp.int32),
            pltpu.SemaphoreType.DMA,
        ],
    )(features, mask_values, hpk, hmk, hcnt)

    out_feats = out_f[:M]
    out_coords = out_c.reshape(3, MP)[:, :M].T
    return out_feats, out_coords


# speedup vs baseline: 1.5660x; 1.0058x over previous
import functools

import jax
import jax.numpy as jnp
from jax import lax
from jax.experimental import pallas as pl
from jax.experimental.pallas import tpu as pltpu
from jax.experimental.pallas import tpu_sc as plsc

B_, H_, W_ = 4, 128, 128
NC, NS = 2, 16
NW = NC * NS
NCELL = B_ * H_ * W_
ROUNDS = 8
BUCK = NCELL // ROUNDS
LBUCK = 13
OWN = BUCK // NW
LOWN = 8
DUMP = OWN
PSHIFT = 17
PMASK = (1 << PSHIFT) - 1
LISTCAP = 4096
RING = 512
FB = 32
GB = 128


def _ceil_to(x, m):
    return (x + m - 1) // m * m


def _k1_body(PT, MT, MP, C,
             cb, cy, cx, mb, my, mx, strd,
             out_c, hpk, hcnt,
             cc0, cc1, cc2, pcell_v, strd_v, cnt_v, cppk):
    PC = PT // 2
    MC = MT // 2
    c = lax.axis_index("c")
    s_id = lax.axis_index("s")
    w = c * NS + s_id
    pbase = w * PT
    mbase = w * MT
    iota = lax.broadcasted_iota(jnp.int32, (16,), 0)
    i32 = jnp.int32

    pltpu.sync_copy(strd.at[pl.ds(0, 16)], strd_v)
    stride_vec = strd_v[pl.ds(0, 16)]

    for q in range(2):
        off = q * PC
        pltpu.sync_copy(cb.at[pl.ds(pbase + off, PC)], cc0.at[pl.ds(0, PC)])
        pltpu.sync_copy(cy.at[pl.ds(pbase + off, PC)], cc1.at[pl.ds(0, PC)])
        pltpu.sync_copy(cx.at[pl.ds(pbase + off, PC)], cc2.at[pl.ds(0, PC)])

        def pcells(i, _, off=off):
            b = cc0[pl.ds(i * 16, 16)]
            y = cc1[pl.ds(i * 16, 16)]
            x = cc2[pl.ds(i * 16, 16)]
            yy = lax.div(y, stride_vec)
            xx = lax.div(x, stride_vec)
            ok = ((b >= 0) & (b < B_) & (yy >= 0) & (yy < H_)
                  & (xx >= 0) & (xx < W_))
            cell = (b * H_ + yy) * W_ + xx
            pcell_v[pl.ds(off + i * 16, 16)] = jnp.where(ok, cell, -1)
            return 0
        lax.fori_loop(0, PC // 16, pcells, 0)

    for q in range(2):
        off = q * MC
        pltpu.sync_copy(mb.at[pl.ds(mbase + off, MC)], cc0.at[pl.ds(0, MC)])
        pltpu.sync_copy(my.at[pl.ds(mbase + off, MC)], cc1.at[pl.ds(0, MC)])
        pltpu.sync_copy(mx.at[pl.ds(mbase + off, MC)], cc2.at[pl.ds(0, MC)])
        pltpu.sync_copy(cc0.at[pl.ds(0, MC)],
                        out_c.at[pl.ds(mbase + off, MC)])

        def scale12(i, _):
            cc1[pl.ds(i * 16, 16)] = cc1[pl.ds(i * 16, 16)] * stride_vec
            cc2[pl.ds(i * 16, 16)] = cc2[pl.ds(i * 16, 16)] * stride_vec
            return 0
        lax.fori_loop(0, MC // 16, scale12, 0)
        pltpu.sync_copy(cc1.at[pl.ds(0, MC)],
                        out_c.at[pl.ds(MP + mbase + off, MC)])
        pltpu.sync_copy(cc2.at[pl.ds(0, MC)],
                        out_c.at[pl.ds(2 * MP + mbase + off, MC)])

    posv = jnp.zeros((16,), i32)
    for sl in range(ROUNDS):
        def scan(i, pv):
            cell = pcell_v[pl.ds(i * 16, 16)]
            hit = (cell >> LBUCK) == sl
            pos = pv + plsc.cumsum(hit.astype(i32)) - 1
            val = ((cell & (BUCK - 1)) << PSHIFT) | (pbase + i * 16 + iota)
            plsc.store_scatter(cppk, [pos], val, mask=hit)
            return pv + plsc.all_reduce_population_count(hit)
        posv2 = lax.fori_loop(0, PT // 16, scan, posv)
        plsc.store_scatter(cnt_v, [jnp.full((16,), sl, i32)], posv2 - posv)
        padv = (256 - (posv2 & 255)) & 255

        def pad16(k, _):
            idx = posv2 + k * 16 + iota
            plsc.store_scatter(cppk, [idx], jnp.full((16,), -1, i32),
                               mask=(k * 16 + iota) < padv)
            return 0
        lax.fori_loop(0, 16, pad16, 0)
        posv = posv2 + padv

    pltpu.sync_copy(cppk, hpk.at[pl.ds(w * LISTCAP, LISTCAP)])
    pltpu.sync_copy(cnt_v, hcnt.at[pl.ds(w * 16, 16)])


def _k2_body(C,
             feat, hpk, hcnt,
             dgrid,
             cnts_in, inb, inb2, ring_g, ring_c, frows, dense,
             sem, sem2):
    c = lax.axis_index("c")
    s_id = lax.axis_index("s")
    w = c * NS + s_id
    iota = lax.broadcasted_iota(jnp.int32, (16,), 0)
    i32 = jnp.int32
    NEG = jnp.int32(-(2**31) + 1)

    pltpu.sync_copy(hcnt, cnts_in)

    def issue_pt(g):
        pltpu.async_copy(feat.at[ring_g.at[(g >> 5) & 15]],
                         frows.at[pl.ds(((g >> 5) & 1) * FB, FB)], sem2)

    def process_pt(_, fired):
        slot = ((fired >> 5) & 1) * FB
        pltpu.make_async_copy(feat.at[pl.ds(0, FB)],
                              frows.at[pl.ds(slot, FB)], sem2).wait()

        def acc_row(rr, _):
            base = (fired + rr) & (RING - 1)
            oc = ring_c[pl.ds(base, 16)][0]
            for j in range(C // 16):
                d = dense[oc, pl.ds(j * 16, 16)]
                dense[oc, pl.ds(j * 16, 16)] = (
                    d + frows[slot + rr, pl.ds(j * 16, 16)])
            return 0
        lax.fori_loop(0, FB, acc_row, 0)
        return fired + FB

    def round_body(r, _):
        def zd(k, _):
            for u in range(8):
                dense[(k * 8 + u) >> 4,
                      pl.ds(((k * 8 + u) & 15) * 16, 16)] = (
                    jnp.zeros((16,), jnp.float32))
            return 0
        lax.fori_loop(0, OWN * (C // 16) // 8, zd, 0)

        def seg(t):
            row0 = cnts_in[pl.ds(t * 16, 16)]
            nchv = (row0 + 255) >> 8
            lo_ch = jnp.sum(jnp.where(iota < r, nchv, 0))
            cnt = jnp.max(jnp.where(iota == r, row0, NEG))
            return t * (LISTCAP // 256) + lo_ch, cnt

        def scan_chunk(buf, cnt_left, cv):
            nv = (jnp.minimum(cnt_left, 256) + 15) >> 4

            def scan16(i, cv):
                pk = buf[pl.ds(i * 16, 16)]
                lcell = pk >> PSHIFT
                gidx = pk & PMASK
                hit = (lcell >> LOWN) == w
                pos = cv + plsc.cumsum(hit.astype(i32)) - 1
                plsc.store_scatter(ring_g, [(pos >> 5) & 15, pos & 31],
                                   gidx, mask=hit)
                plsc.store_scatter(ring_c, [pos & (RING - 1)],
                                   lcell & (OWN - 1), mask=hit)
                return cv + plsc.all_reduce_population_count(hit)
            return lax.fori_loop(0, nv, scan16, cv)

        def pump(cv, fired, giss):
            mc = jnp.max(cv)
            fired = lax.fori_loop(0, (giss - fired) >> 5, process_pt, fired)
            ncatch = (jnp.maximum(mc - fired - 192, 0) + 31) >> 5

            def catch(_, c2):
                fired, giss = c2
                issue_pt(giss)
                return (process_pt(0, fired), giss + FB)
            fired, giss = lax.fori_loop(0, ncatch, catch, (fired, giss))

            def iss(_, g):
                issue_pt(g)
                return g + FB
            giss = lax.fori_loop(0, jnp.minimum((mc - giss) >> 5, 2),
                                 iss, giss)
            return fired, giss

        for tp in range(4):
            chp, _ = seg(tp)
            pltpu.async_copy(hpk.at[pl.ds(chp * 256, 256)],
                             inb.at[pl.ds(tp * 256, 256)], sem)

        def src_loop(t, carry):
            cv, fired, giss = carry
            _, cnt = seg(t)
            nch = (cnt + 255) >> 8
            pltpu.make_async_copy(hpk.at[pl.ds(0, 256)],
                                  inb.at[pl.ds((t & 3) * 256, 256)],
                                  sem).wait()
            cv = scan_chunk(inb.at[pl.ds((t & 3) * 256, 256)], cnt, cv)
            tn = jnp.minimum(t + 4, NW - 1)
            chn, _ = seg(tn)
            pltpu.async_copy(hpk.at[pl.ds(chn * 256, 256)],
                             inb.at[pl.ds((t & 3) * 256, 256)], sem)
            fired, giss = pump(cv, fired, giss)

            def chunk(q, c2):
                cv, fired, giss = c2
                ch, _ = seg(t)
                pltpu.sync_copy(hpk.at[pl.ds((ch + q) * 256, 256)], inb2)
                cv = scan_chunk(inb2, cnt - q * 256, cv)
                fired, giss = pump(cv, fired, giss)
                return (cv, fired, giss)
            return lax.fori_loop(1, nch, chunk, (cv, fired, giss))
        cv, fired, giss = lax.fori_loop(
            0, NW, src_loop, (jnp.zeros((16,), i32), i32(0), i32(0)))
        for tp in range(4):
            pltpu.make_async_copy(hpk.at[pl.ds(0, 256)],
                                  inb.at[pl.ds(tp * 256, 256)], sem).wait()
        mcnt = jnp.max(cv)
        mstart = mcnt & ~31
        rowsp = jnp.full((16,), (mstart >> 5) & 15, i32)
        for k in range(FB // 16):
            padmask = (mstart + k * 16 + iota) >= mcnt
            colv = (mstart + k * 16 + iota) & 31
            plsc.store_scatter(ring_g, [rowsp, colv],
                               jnp.zeros((16,), i32), mask=padmask)
            plsc.store_scatter(ring_c,
                               [(mstart & (RING - 1)) + k * 16 + iota],
                               jnp.full((16,), DUMP, i32), mask=padmask)
        fired = lax.fori_loop(0, (giss - fired) >> 5, process_pt, fired)
        mc2 = (mcnt + FB - 1) & ~31

        def last(_, c2):
            fired, giss = c2
            issue_pt(giss)
            return (process_pt(0, fired), giss + FB)
        lax.fori_loop(0, (mc2 - giss) >> 5, last, (fired, giss))

        pltpu.sync_copy(dense.at[pl.ds(0, OWN), :],
                        dgrid.at[pl.ds(r * BUCK + w * OWN, OWN), :])
        return 0

    lax.fori_loop(0, ROUNDS, round_body, 0)


def _k3_body(M, MT, C,
             mval, mb, my, mx, dgrid,
             out_f,
             cc0, cc1, cc2, cidx, gidx, grows, mvrows, sem):
    c = lax.axis_index("c")
    s_id = lax.axis_index("s")
    w = c * NS + s_id
    mbase = w * MT
    iota = lax.broadcasted_iota(jnp.int32, (16,), 0)
    i32 = jnp.int32

    MC = MT // 2
    for q in range(2):
        off = q * MC
        pltpu.sync_copy(mb.at[pl.ds(mbase + off, MC)], cc0.at[pl.ds(0, MC)])
        pltpu.sync_copy(my.at[pl.ds(mbase + off, MC)], cc1.at[pl.ds(0, MC)])
        pltpu.sync_copy(mx.at[pl.ds(mbase + off, MC)], cc2.at[pl.ds(0, MC)])

        def mcells(i, _, off=off):
            cell = ((cc0[pl.ds(i * 16, 16)] * H_ + cc1[pl.ds(i * 16, 16)])
                    * W_ + cc2[pl.ds(i * 16, 16)])
            cell = jnp.clip(cell, 0, NCELL - 1)
            cidx[pl.ds(off + i * 16, 16)] = cell
            midx = jnp.minimum(mbase + off + i * 16 + iota, M - 1)
            gidx[pl.ds(off + i * 16, 16)] = midx
            return 0
        lax.fori_loop(0, MC // 16, mcells, 0)

    def batch(q, _):
        pltpu.async_copy(dgrid.at[cidx.at[pl.ds(q * GB, GB)]], grows, sem)
        pltpu.make_async_copy(dgrid.at[pl.ds(0, GB)], grows, sem).wait()
        pltpu.sync_copy(mval.at[gidx.at[pl.ds(q * GB, GB)]], mvrows)

        def mul(t, _):
            i = t >> 4
            jj = t & 15
            grows[i, pl.ds(jj * 16, 16)] = (
                grows[i, pl.ds(jj * 16, 16)]
                * mvrows[i, pl.ds(jj * 16, 16)])
            return 0
        lax.fori_loop(0, GB * (C // 16), mul, 0)
        pltpu.sync_copy(grows, out_f.at[pl.ds(mbase + q * GB, GB), :])
        return 0
    lax.fori_loop(0, MT // GB, batch, 0)


def kernel(features, mask_values, coords_b, coords_y, coords_x,
           mask_b, mask_y, mask_x, stride):
    N, C = features.shape
    M = mask_values.shape[0]
    PT = _ceil_to(-(-N // NW), 32)
    MT = _ceil_to(-(-M // NW), 256)
    NP = NW * PT
    MP = NW * MT
    BIG = jnp.int32(1 << 20)

    cb = jnp.pad(coords_b, (0, NP - N))
    cy = jnp.pad(coords_y, (0, NP - N), constant_values=BIG)
    cx = jnp.pad(coords_x, (0, NP - N))
    mb = jnp.pad(mask_b, (0, MP - M))
    my = jnp.pad(mask_y, (0, MP - M), constant_values=BIG)
    mx = jnp.pad(mask_x, (0, MP - M))
    strd = jnp.full((16,), stride, jnp.int32)

    mesh = plsc.VectorSubcoreMesh(core_axis_name="c", subcore_axis_name="s")
    cp = pltpu.CompilerParams(needs_layout_passes=False)

    out_c, hpk, hcnt = pl.kernel(
        functools.partial(_k1_body, PT, MT, MP, C),
        out_type=(
            jax.ShapeDtypeStruct((3 * MP,), jnp.int32),
            jax.ShapeDtypeStruct((NW * LISTCAP,), jnp.int32),
            jax.ShapeDtypeStruct((NW * 16,), jnp.int32),
        ),
        mesh=mesh,
        compiler_params=cp,
        scratch_types=[
            pltpu.VMEM((PT // 2,), jnp.int32),
            pltpu.VMEM((PT // 2,), jnp.int32),
            pltpu.VMEM((PT // 2,), jnp.int32),
            pltpu.VMEM((PT,), jnp.int32),
            pltpu.VMEM((16,), jnp.int32),
            pltpu.VMEM((16,), jnp.int32),
            pltpu.VMEM((LISTCAP,), jnp.int32),
        ],
    )(cb, cy, cx, mb, my, mx, strd)

    dgrid, = pl.kernel(
        functools.partial(_k2_body, C),
        out_type=(
            jax.ShapeDtypeStruct((NCELL, C), jnp.float32),
        ),
        mesh=mesh,
        compiler_params=cp,
        scratch_types=[
            pltpu.VMEM((NW * 16,), jnp.int32),
            pltpu.VMEM((1024,), jnp.int32),
            pltpu.VMEM((256,), jnp.int32),
            pltpu.VMEM((16, 32), jnp.int32),
            pltpu.VMEM((RING + 16,), jnp.int32),
            pltpu.VMEM((2 * FB, C), jnp.float32),
            pltpu.VMEM((OWN + 8, C), jnp.float32),
            pltpu.SemaphoreType.DMA,
            pltpu.SemaphoreType.DMA,
        ],
    )(features, hpk, hcnt)

    out_f, = pl.kernel(
        functools.partial(_k3_body, M, MT, C),
        out_type=(
            jax.ShapeDtypeStruct((MP, C), jnp.float32),
        ),
        mesh=mesh,
        compiler_params=cp,
        scratch_types=[
            pltpu.VMEM((MT // 2,), jnp.int32),
            pltpu.VMEM((MT // 2,), jnp.int32),
            pltpu.VMEM((MT // 2,), jnp.int32),
            pltpu.VMEM((MT,), jnp.int32),
            pltpu.VMEM((MT,), jnp.int32),
            pltpu.VMEM((GB, C), jnp.float32),
            pltpu.VMEM((GB, C), jnp.float32),
            pltpu.SemaphoreType.DMA,
        ],
    )(mask_values, mb, my, mx, dgrid)

    out_feats = out_f[:M]
    out_coords = out_c.reshape(3, MP)[:, :M].T
    return out_feats, out_coords


# speedup vs baseline: 1.9249x; 1.2292x over previous
import functools

import jax
import jax.numpy as jnp
from jax import lax
from jax.experimental import pallas as pl
from jax.experimental.pallas import tpu as pltpu
from jax.experimental.pallas import tpu_sc as plsc

B_, H_, W_ = 4, 128, 128
NC, NS = 2, 16
NW = NC * NS
NCELL = B_ * H_ * W_
ROUNDS = 8
BUCK = NCELL // ROUNDS
LBUCK = 13
OWN = BUCK // NW
LOWN = 8
DUMP = OWN
PSHIFT = 17
PMASK = (1 << PSHIFT) - 1
LISTCAP = 4096
RING = 512
FB = 32
GB = 64


def _ceil_to(x, m):
    return (x + m - 1) // m * m


def _k1_body(PT, MT, MP, C,
             cb, cy, cx, mb, my, mx, strd,
             out_c, hpk, hcnt,
             cc0, cc1, cc2, pcell_v, strd_v, cnt_v, cppk):
    PC = PT // 2
    MC = MT // 2
    c = lax.axis_index("c")
    s_id = lax.axis_index("s")
    w = c * NS + s_id
    pbase = w * PT
    mbase = w * MT
    iota = lax.broadcasted_iota(jnp.int32, (16,), 0)
    i32 = jnp.int32

    pltpu.sync_copy(strd.at[pl.ds(0, 16)], strd_v)
    stride_vec = strd_v[pl.ds(0, 16)]

    for q in range(2):
        off = q * PC
        pltpu.sync_copy(cb.at[pl.ds(pbase + off, PC)], cc0.at[pl.ds(0, PC)])
        pltpu.sync_copy(cy.at[pl.ds(pbase + off, PC)], cc1.at[pl.ds(0, PC)])
        pltpu.sync_copy(cx.at[pl.ds(pbase + off, PC)], cc2.at[pl.ds(0, PC)])

        def pcells(i, _, off=off):
            b = cc0[pl.ds(i * 16, 16)]
            y = cc1[pl.ds(i * 16, 16)]
            x = cc2[pl.ds(i * 16, 16)]
            yy = lax.div(y, stride_vec)
            xx = lax.div(x, stride_vec)
            ok = ((b >= 0) & (b < B_) & (yy >= 0) & (yy < H_)
                  & (xx >= 0) & (xx < W_))
            cell = (b * H_ + yy) * W_ + xx
            pcell_v[pl.ds(off + i * 16, 16)] = jnp.where(ok, cell, -1)
            return 0
        lax.fori_loop(0, PC // 16, pcells, 0)

    for q in range(2):
        off = q * MC
        pltpu.sync_copy(mb.at[pl.ds(mbase + off, MC)], cc0.at[pl.ds(0, MC)])
        pltpu.sync_copy(my.at[pl.ds(mbase + off, MC)], cc1.at[pl.ds(0, MC)])
        pltpu.sync_copy(mx.at[pl.ds(mbase + off, MC)], cc2.at[pl.ds(0, MC)])
        pltpu.sync_copy(cc0.at[pl.ds(0, MC)],
                        out_c.at[pl.ds(mbase + off, MC)])

        def scale12(i, _):
            cc1[pl.ds(i * 16, 16)] = cc1[pl.ds(i * 16, 16)] * stride_vec
            cc2[pl.ds(i * 16, 16)] = cc2[pl.ds(i * 16, 16)] * stride_vec
            return 0
        lax.fori_loop(0, MC // 16, scale12, 0)
        pltpu.sync_copy(cc1.at[pl.ds(0, MC)],
                        out_c.at[pl.ds(MP + mbase + off, MC)])
        pltpu.sync_copy(cc2.at[pl.ds(0, MC)],
                        out_c.at[pl.ds(2 * MP + mbase + off, MC)])

    posv = jnp.zeros((16,), i32)
    for sl in range(ROUNDS):
        def scan(i, pv):
            cell = pcell_v[pl.ds(i * 16, 16)]
            hit = (cell >> LBUCK) == sl
            pos = pv + plsc.cumsum(hit.astype(i32)) - 1
            val = ((cell & (BUCK - 1)) << PSHIFT) | (pbase + i * 16 + iota)
            plsc.store_scatter(cppk, [pos], val, mask=hit)
            return pv + plsc.all_reduce_population_count(hit)
        posv2 = lax.fori_loop(0, PT // 16, scan, posv)
        plsc.store_scatter(cnt_v, [jnp.full((16,), sl, i32)], posv2 - posv)
        padv = (256 - (posv2 & 255)) & 255

        def pad16(k, _):
            idx = posv2 + k * 16 + iota
            plsc.store_scatter(cppk, [idx], jnp.full((16,), -1, i32),
                               mask=(k * 16 + iota) < padv)
            return 0
        lax.fori_loop(0, 16, pad16, 0)
        posv = posv2 + padv

    pltpu.sync_copy(cppk, hpk.at[pl.ds(w * LISTCAP, LISTCAP)])
    pltpu.sync_copy(cnt_v, hcnt.at[pl.ds(w * 16, 16)])


def _k2_body(C,
             feat, hpk, hcnt,
             dgrid,
             cnts_in, inb, inb2, ring_g, ring_c, frows, dense,
             sem, sem2):
    c = lax.axis_index("c")
    s_id = lax.axis_index("s")
    w = c * NS + s_id
    iota = lax.broadcasted_iota(jnp.int32, (16,), 0)
    i32 = jnp.int32
    NEG = jnp.int32(-(2**31) + 1)

    pltpu.sync_copy(hcnt, cnts_in)

    def issue_pt(g):
        pltpu.async_copy(feat.at[ring_g.at[(g >> 5) & 15]],
                         frows.at[pl.ds(((g >> 5) & 1) * FB, FB)], sem2)

    def process_pt(_, fired):
        slot = ((fired >> 5) & 1) * FB
        pltpu.make_async_copy(feat.at[pl.ds(0, FB)],
                              frows.at[pl.ds(slot, FB)], sem2).wait()

        def acc_row(rr, _):
            base = (fired + rr) & (RING - 1)
            oc = ring_c[pl.ds(base, 16)][0]
            for j in range(C // 16):
                d = dense[oc, pl.ds(j * 16, 16)]
                dense[oc, pl.ds(j * 16, 16)] = (
                    d + frows[slot + rr, pl.ds(j * 16, 16)])
            return 0
        lax.fori_loop(0, FB, acc_row, 0)
        return fired + FB

    def round_body(r, _):
        def zd(k, _):
            for u in range(8):
                dense[(k * 8 + u) >> 4,
                      pl.ds(((k * 8 + u) & 15) * 16, 16)] = (
                    jnp.zeros((16,), jnp.float32))
            return 0
        lax.fori_loop(0, OWN * (C // 16) // 8, zd, 0)

        def seg(t):
            row0 = cnts_in[pl.ds(t * 16, 16)]
            nchv = (row0 + 255) >> 8
            lo_ch = jnp.sum(jnp.where(iota < r, nchv, 0))
            cnt = jnp.max(jnp.where(iota == r, row0, NEG))
            return t * (LISTCAP // 256) + lo_ch, cnt

        def scan_chunk(buf, cnt_left, cv):
            nv = (jnp.minimum(cnt_left, 256) + 15) >> 4

            def scan16(i, cv):
                pk = buf[pl.ds(i * 16, 16)]
                lcell = pk >> PSHIFT
                gidx = pk & PMASK
                hit = (lcell >> LOWN) == w
                pos = cv + plsc.cumsum(hit.astype(i32)) - 1
                plsc.store_scatter(ring_g, [(pos >> 5) & 15, pos & 31],
                                   gidx, mask=hit)
                plsc.store_scatter(ring_c, [pos & (RING - 1)],
                                   lcell & (OWN - 1), mask=hit)
                return cv + plsc.all_reduce_population_count(hit)
            return lax.fori_loop(0, nv, scan16, cv)

        def pump(cv, fired, giss):
            mc = jnp.max(cv)
            fired = lax.fori_loop(0, (giss - fired) >> 5, process_pt, fired)
            ncatch = (jnp.maximum(mc - fired - 192, 0) + 31) >> 5

            def catch(_, c2):
                fired, giss = c2
                issue_pt(giss)
                return (process_pt(0, fired), giss + FB)
            fired, giss = lax.fori_loop(0, ncatch, catch, (fired, giss))

            def iss(_, g):
                issue_pt(g)
                return g + FB
            giss = lax.fori_loop(0, jnp.minimum((mc - giss) >> 5, 2),
                                 iss, giss)
            return fired, giss

        for tp in range(4):
            chp, _ = seg(tp)
            pltpu.async_copy(hpk.at[pl.ds(chp * 256, 256)],
                             inb.at[pl.ds(tp * 256, 256)], sem)

        def src_loop(t, carry):
            cv, fired, giss = carry
            _, cnt = seg(t)
            nch = (cnt + 255) >> 8
            pltpu.make_async_copy(hpk.at[pl.ds(0, 256)],
                                  inb.at[pl.ds((t & 3) * 256, 256)],
                                  sem).wait()
            cv = scan_chunk(inb.at[pl.ds((t & 3) * 256, 256)], cnt, cv)
            tn = jnp.minimum(t + 4, NW - 1)
            chn, _ = seg(tn)
            pltpu.async_copy(hpk.at[pl.ds(chn * 256, 256)],
                             inb.at[pl.ds((t & 3) * 256, 256)], sem)
            fired, giss = pump(cv, fired, giss)

            def chunk(q, c2):
                cv, fired, giss = c2
                ch, _ = seg(t)
                pltpu.sync_copy(hpk.at[pl.ds((ch + q) * 256, 256)], inb2)
                cv = scan_chunk(inb2, cnt - q * 256, cv)
                fired, giss = pump(cv, fired, giss)
                return (cv, fired, giss)
            return lax.fori_loop(1, nch, chunk, (cv, fired, giss))
        cv, fired, giss = lax.fori_loop(
            0, NW, src_loop, (jnp.zeros((16,), i32), i32(0), i32(0)))
        for tp in range(4):
            pltpu.make_async_copy(hpk.at[pl.ds(0, 256)],
                                  inb.at[pl.ds(tp * 256, 256)], sem).wait()
        mcnt = jnp.max(cv)
        mstart = mcnt & ~31
        rowsp = jnp.full((16,), (mstart >> 5) & 15, i32)
        for k in range(FB // 16):
            padmask = (mstart + k * 16 + iota) >= mcnt
            colv = (mstart + k * 16 + iota) & 31
            plsc.store_scatter(ring_g, [rowsp, colv],
                               jnp.zeros((16,), i32), mask=padmask)
            plsc.store_scatter(ring_c,
                               [(mstart & (RING - 1)) + k * 16 + iota],
                               jnp.full((16,), DUMP, i32), mask=padmask)
        fired = lax.fori_loop(0, (giss - fired) >> 5, process_pt, fired)
        mc2 = (mcnt + FB - 1) & ~31

        def last(_, c2):
            fired, giss = c2
            issue_pt(giss)
            return (process_pt(0, fired), giss + FB)
        lax.fori_loop(0, (mc2 - giss) >> 5, last, (fired, giss))

        pltpu.sync_copy(dense.at[pl.ds(0, OWN), :],
                        dgrid.at[pl.ds(r * BUCK + w * OWN, OWN), :])
        return 0

    lax.fori_loop(0, ROUNDS, round_body, 0)


def _k3_body(M, MT, C,
             mval, mb, my, mx, dgrid,
             out_f,
             cc0, cc1, cc2, cidx, gidx, grows, mvrows, sem):
    c = lax.axis_index("c")
    s_id = lax.axis_index("s")
    w = c * NS + s_id
    mbase = w * MT
    iota = lax.broadcasted_iota(jnp.int32, (16,), 0)
    i32 = jnp.int32

    MC = MT // 2
    for q in range(2):
        off = q * MC
        pltpu.sync_copy(mb.at[pl.ds(mbase + off, MC)], cc0.at[pl.ds(0, MC)])
        pltpu.sync_copy(my.at[pl.ds(mbase + off, MC)], cc1.at[pl.ds(0, MC)])
        pltpu.sync_copy(mx.at[pl.ds(mbase + off, MC)], cc2.at[pl.ds(0, MC)])

        def mcells(i, _, off=off):
            cell = ((cc0[pl.ds(i * 16, 16)] * H_ + cc1[pl.ds(i * 16, 16)])
                    * W_ + cc2[pl.ds(i * 16, 16)])
            cell = jnp.clip(cell, 0, NCELL - 1)
            cidx[pl.ds(off + i * 16, 16)] = cell
            midx = jnp.minimum(mbase + off + i * 16 + iota, M - 1)
            gidx[pl.ds(off + i * 16, 16)] = midx
            return 0
        lax.fori_loop(0, MC // 16, mcells, 0)

    NB = MT // GB

    def issue(q, slot):
        pltpu.async_copy(dgrid.at[cidx.at[pl.ds(q * GB, GB)]],
                         grows.at[pl.ds(slot * GB, GB)], sem)
        pltpu.async_copy(mval.at[gidx.at[pl.ds(q * GB, GB)]],
                         mvrows.at[pl.ds(slot * GB, GB)], sem)

    issue(0, 0)

    def batch(q, _):
        slot = (q & 1) * GB
        pltpu.make_async_copy(dgrid.at[pl.ds(0, GB)],
                              grows.at[pl.ds(slot, GB)], sem).wait()
        pltpu.make_async_copy(mval.at[pl.ds(0, GB)],
                              mvrows.at[pl.ds(slot, GB)], sem).wait()
        issue(jnp.minimum(q + 1, NB - 1), (q + 1) & 1)

        def mul(i, _):
            for jj in range(C // 16):
                grows[slot + i, pl.ds(jj * 16, 16)] = (
                    grows[slot + i, pl.ds(jj * 16, 16)]
                    * mvrows[slot + i, pl.ds(jj * 16, 16)])
            return 0
        lax.fori_loop(0, GB, mul, 0)
        pltpu.sync_copy(grows.at[pl.ds(slot, GB)],
                        out_f.at[pl.ds(mbase + q * GB, GB), :])
        return 0
    lax.fori_loop(0, NB, batch, 0)
    pltpu.make_async_copy(dgrid.at[pl.ds(0, GB)],
                          grows.at[pl.ds(0, GB)], sem).wait()
    pltpu.make_async_copy(mval.at[pl.ds(0, GB)],
                          mvrows.at[pl.ds(0, GB)], sem).wait()


def kernel(features, mask_values, coords_b, coords_y, coords_x,
           mask_b, mask_y, mask_x, stride):
    N, C = features.shape
    M = mask_values.shape[0]
    PT = _ceil_to(-(-N // NW), 32)
    MT = _ceil_to(-(-M // NW), 256)
    NP = NW * PT
    MP = NW * MT
    BIG = jnp.int32(1 << 20)

    cb = jnp.pad(coords_b, (0, NP - N))
    cy = jnp.pad(coords_y, (0, NP - N), constant_values=BIG)
    cx = jnp.pad(coords_x, (0, NP - N))
    mb = jnp.pad(mask_b, (0, MP - M))
    my = jnp.pad(mask_y, (0, MP - M), constant_values=BIG)
    mx = jnp.pad(mask_x, (0, MP - M))
    strd = jnp.full((16,), stride, jnp.int32)

    mesh = plsc.VectorSubcoreMesh(core_axis_name="c", subcore_axis_name="s")
    cp = pltpu.CompilerParams(needs_layout_passes=False)

    out_c, hpk, hcnt = pl.kernel(
        functools.partial(_k1_body, PT, MT, MP, C),
        out_type=(
            jax.ShapeDtypeStruct((3 * MP,), jnp.int32),
            jax.ShapeDtypeStruct((NW * LISTCAP,), jnp.int32),
            jax.ShapeDtypeStruct((NW * 16,), jnp.int32),
        ),
        mesh=mesh,
        compiler_params=cp,
        scratch_types=[
            pltpu.VMEM((PT // 2,), jnp.int32),
            pltpu.VMEM((PT // 2,), jnp.int32),
            pltpu.VMEM((PT // 2,), jnp.int32),
            pltpu.VMEM((PT,), jnp.int32),
            pltpu.VMEM((16,), jnp.int32),
            pltpu.VMEM((16,), jnp.int32),
            pltpu.VMEM((LISTCAP,), jnp.int32),
        ],
    )(cb, cy, cx, mb, my, mx, strd)

    dgrid, = pl.kernel(
        functools.partial(_k2_body, C),
        out_type=(
            jax.ShapeDtypeStruct((NCELL, C), jnp.float32),
        ),
        mesh=mesh,
        compiler_params=cp,
        scratch_types=[
            pltpu.VMEM((NW * 16,), jnp.int32),
            pltpu.VMEM((1024,), jnp.int32),
            pltpu.VMEM((256,), jnp.int32),
            pltpu.VMEM((16, 32), jnp.int32),
            pltpu.VMEM((RING + 16,), jnp.int32),
            pltpu.VMEM((2 * FB, C), jnp.float32),
            pltpu.VMEM((OWN + 8, C), jnp.float32),
            pltpu.SemaphoreType.DMA,
            pltpu.SemaphoreType.DMA,
        ],
    )(features, hpk, hcnt)

    out_f, = pl.kernel(
        functools.partial(_k3_body, M, MT, C),
        out_type=(
            jax.ShapeDtypeStruct((MP, C), jnp.float32),
        ),
        mesh=mesh,
        compiler_params=cp,
        scratch_types=[
            pltpu.VMEM((MT // 2,), jnp.int32),
            pltpu.VMEM((MT // 2,), jnp.int32),
            pltpu.VMEM((MT // 2,), jnp.int32),
            pltpu.VMEM((MT,), jnp.int32),
            pltpu.VMEM((MT,), jnp.int32),
            pltpu.VMEM((2 * GB, C), jnp.float32),
            pltpu.VMEM((2 * GB, C), jnp.float32),
            pltpu.SemaphoreType.DMA,
        ],
    )(mask_values, mb, my, mx, dgrid)

    out_feats = out_f[:M]
    out_coords = out_c.reshape(3, MP)[:, :M].T
    return out_feats, out_coords
